# baseline (speedup 1.0000x reference)
import functools

import jax
import jax.numpy as jnp
from jax import lax
from jax.experimental import pallas as pl
from jax.experimental.pallas import tpu as pltpu
from jax.experimental.pallas import tpu_sc as plsc

_THRESHOLD = 0.85

_NC = 2
_NS = 16
_L = 16
_NW = _NC * _NS
_CHUNK = 64
_UNROLL = 8

_RB = 512
_SC_ROWS = 22528


@functools.lru_cache(maxsize=None)
def _make_sc_kernel(rows_all: int, k: int, row0: int, rows_sc: int):
    rows_per_w = rows_sc // _NW
    n_chunks = rows_per_w // _CHUNK
    mesh = plsc.VectorSubcoreMesh(core_axis_name="c", subcore_axis_name="s")

    @functools.partial(
        pl.kernel,
        mesh=mesh,
        out_type=(
            jax.ShapeDtypeStruct((rows_sc,), jnp.float32),
            jax.ShapeDtypeStruct((rows_sc,), jnp.int32),
        ),
        scratch_types=[
            pltpu.VMEM((_CHUNK, k), jnp.float32),
            pltpu.VMEM((rows_per_w,), jnp.float32),
            pltpu.VMEM((rows_per_w,), jnp.int32),
        ],
        compiler_params=pltpu.CompilerParams(
            use_tc_tiling_on_sc=False, needs_layout_passes=False),
    )
    def sc_kernel(sim_hbm, score_hbm, idx_hbm, buf, acc_s, acc_i):
        wid = lax.axis_index("s") * _NC + lax.axis_index("c")
        out_base = wid * rows_per_w
        in_base = row0 + out_base
        lane = lax.iota(jnp.int32, _L)

        def chunk_body(g, carry):
            pltpu.sync_copy(sim_hbm.at[pl.ds(in_base + g * _CHUNK, _CHUNK)],
                            buf)
            for r in range(0, _CHUNK, _L):
                row_ids = lane + r

                def step(_, st):
                    vmaxs, vidxs, kbase = st
                    vmaxs, vidxs = list(vmaxs), list(vidxs)
                    for u in range(_UNROLL):
                        kv = kbase + u
                        v = plsc.load_gather(buf, [row_ids, kv])
                        pred = v > vmaxs[u]
                        vmaxs[u] = jnp.where(pred, v, vmaxs[u])
                        vidxs[u] = jnp.where(pred, kv, vidxs[u])
                    return tuple(vmaxs), tuple(vidxs), kbase + _UNROLL

                init = (
                    tuple(jnp.full((_L,), -jnp.inf, jnp.float32)
                          for _ in range(_UNROLL)),
                    tuple(jnp.zeros((_L,), jnp.int32)
                          for _ in range(_UNROLL)),
                    jnp.zeros((_L,), jnp.int32),
                )
                vmaxs, vidxs, _ = lax.fori_loop(0, k // _UNROLL, step, init)
                vmax, vidx = vmaxs[0], vidxs[0]
                for u in range(1, _UNROLL):
                    vb, ib = vmaxs[u], vidxs[u]
                    better = (vb > vmax) | ((vb == vmax) & (ib < vidx))
                    vmax = jnp.where(better, vb, vmax)
                    vidx = jnp.where(better, ib, vidx)
                off = g * _CHUNK + r
                acc_s[pl.ds(off, _L)] = vmax - _THRESHOLD
                acc_i[pl.ds(off, _L)] = vidx
            return carry

        lax.fori_loop(0, n_chunks, chunk_body, 0)
        pltpu.sync_copy(acc_s, score_hbm.at[pl.ds(out_base, rows_per_w)])
        pltpu.sync_copy(acc_i, idx_hbm.at[pl.ds(out_base, rows_per_w)])

    return sc_kernel


@functools.lru_cache(maxsize=None)
def _make_tc_kernel(nb_all: int, k: int, nb: int):
    def body(x_ref, s_ref, i_ref):
        x = x_ref[...]
        m = jnp.max(x, axis=1, keepdims=True)
        revf = (k - lax.broadcasted_iota(jnp.int32, (_RB, k), 1)
                ).astype(jnp.float32)
        cand = jnp.where(x == m, revf, jnp.float32(0))
        idx = (jnp.float32(k) - jnp.max(cand, axis=1)).astype(jnp.int32)
        s_ref[0, 0, :] = m[:, 0] - _THRESHOLD
        i_ref[0, 0, :] = idx

    return pl.pallas_call(
        body,
        grid=(nb,),
        in_specs=[pl.BlockSpec((_RB, k), lambda i: (i, 0))],
        out_specs=[
            pl.BlockSpec((1, 1, _RB), lambda i: (i, 0, 0)),
            pl.BlockSpec((1, 1, _RB), lambda i: (i, 0, 0)),
        ],
        out_shape=[
            jax.ShapeDtypeStruct((nb, 1, _RB), jnp.float32),
            jax.ShapeDtypeStruct((nb, 1, _RB), jnp.int32),
        ],
    )


def kernel(importance, similarity, compressed_map):
    b, n, k = similarity.shape
    rows = b * n
    rows_tc = rows - _SC_ROWS
    sim2d = similarity.reshape(rows, k)

    parts_s, parts_i = [], []
    if _SC_ROWS:
        sc_s, sc_i = _make_sc_kernel(rows, k, rows_tc, _SC_ROWS)(sim2d)
    if rows_tc:
        tc_s, tc_i = _make_tc_kernel(rows // _RB, k, rows_tc // _RB)(sim2d)
        parts_s.append(tc_s.reshape(rows_tc))
        parts_i.append(tc_i.reshape(rows_tc))
    if _SC_ROWS:
        parts_s.append(sc_s)
        parts_i.append(sc_i)
    score = jnp.concatenate(parts_s) if len(parts_s) > 1 else parts_s[0]
    idx = jnp.concatenate(parts_i) if len(parts_i) > 1 else parts_i[0]
    return (score.reshape(b, n, 1), idx.reshape(b, n))


# speedup vs baseline: 1.0084x; 1.0084x over previous
import functools

import jax
import jax.numpy as jnp
from jax import lax
from jax.experimental import pallas as pl
from jax.experimental.pallas import tpu as pltpu
from jax.experimental.pallas import tpu_sc as plsc

_THRESHOLD = 0.85

_NC = 2
_NS = 16
_L = 16
_NW = _NC * _NS
_CHUNK = 64
_UNROLL = 8

_RB = 512
_SC_ROWS = 24576


@functools.lru_cache(maxsize=None)
def _make_sc_kernel(rows_all: int, k: int, row0: int, rows_sc: int):
    rows_per_w = rows_sc // _NW
    n_chunks = rows_per_w // _CHUNK
    n_slices = k // _L
    mesh = plsc.VectorSubcoreMesh(core_axis_name="c", subcore_axis_name="s")

    @functools.partial(
        pl.kernel,
        mesh=mesh,
        out_type=(
            jax.ShapeDtypeStruct((rows_sc,), jnp.float32),
            jax.ShapeDtypeStruct((rows_sc,), jnp.int32),
        ),
        scratch_types=[
            pltpu.VMEM((_CHUNK, k), jnp.float32),
            pltpu.VMEM((_CHUNK, k), jnp.float32),
            pltpu.VMEM((rows_per_w,), jnp.float32),
            pltpu.VMEM((rows_per_w,), jnp.int32),
            pltpu.VMEM((_L * 17,), jnp.float32),
            pltpu.VMEM((_L * 17,), jnp.int32),
            pltpu.SemaphoreType.DMA,
            pltpu.SemaphoreType.DMA,
        ],
        compiler_params=pltpu.CompilerParams(
            use_tc_tiling_on_sc=False, needs_layout_passes=False),
    )
    def sc_kernel(sim_hbm, score_hbm, idx_hbm,
                  buf0, buf1, acc_s, acc_i, mtx_v, mtx_i, sem0, sem1):
        wid = lax.axis_index("s") * _NC + lax.axis_index("c")
        out_base = wid * rows_per_w
        in_base = row0 + out_base
        lane = lax.iota(jnp.int32, _L)
        rowptr = lane * 17
        bufs = (buf0, buf1)
        sems = (sem0, sem1)

        def start(g, b):
            pltpu.async_copy(
                sim_hbm.at[pl.ds(in_base + g * _CHUNK, _CHUNK)],
                bufs[b], sems[b])

        def wait(b):
            pltpu.make_async_copy(
                sim_hbm.at[pl.ds(in_base, _CHUNK)], bufs[b], sems[b]).wait()

        def process(buf, g):
            def group_body(grp, carry):
                r0 = grp * _L
                for q in range(0, _L, 4):
                    vmax = [buf[r0 + q + j, pl.ds(0, _L)] for j in range(4)]
                    vidx = [lane for _ in range(4)]
                    col = lane
                    for _s in range(1, n_slices):
                        col = col + _L
                        for j in range(4):
                            v = buf[r0 + q + j, pl.ds(_s * _L, _L)]
                            pred = v > vmax[j]
                            vmax[j] = jnp.where(pred, v, vmax[j])
                            vidx[j] = jnp.where(pred, col, vidx[j])
                    for j in range(4):
                        mtx_v[pl.ds((q + j) * 17, _L)] = vmax[j]
                        mtx_i[pl.ds((q + j) * 17, _L)] = vidx[j]
                m = plsc.load_gather(mtx_v, [rowptr])
                i = plsc.load_gather(mtx_i, [rowptr])
                for c in range(1, _L):
                    vc = plsc.load_gather(mtx_v, [rowptr + c])
                    ic = plsc.load_gather(mtx_i, [rowptr + c])
                    better = (vc > m) | ((vc == m) & (ic < i))
                    m = jnp.where(better, vc, m)
                    i = jnp.where(better, ic, i)
                off = g * _CHUNK + grp * _L
                acc_s[pl.ds(off, _L)] = m - _THRESHOLD
                acc_i[pl.ds(off, _L)] = i
                return carry

            lax.fori_loop(0, _CHUNK // _L, group_body, 0)

        start(0, 0)
        start(1, 1)

        def pair_body(p, carry):
            for b in range(2):
                g = 2 * p + b
                wait(b)
                process(bufs[b], g)

                @pl.when(g + 2 < n_chunks)
                def _():
                    start(g + 2, b)
            return carry

        lax.fori_loop(0, n_chunks // 2, pair_body, 0)
        pltpu.sync_copy(acc_s, score_hbm.at[pl.ds(out_base, rows_per_w)])
        pltpu.sync_copy(acc_i, idx_hbm.at[pl.ds(out_base, rows_per_w)])

    return sc_kernel


@functools.lru_cache(maxsize=None)
def _make_tc_kernel(nb_all: int, k: int, nb: int):
    def body(x_ref, s_ref, i_ref):
        x = x_ref[...]
        m = jnp.max(x, axis=1, keepdims=True)
        revf = (k - lax.broadcasted_iota(jnp.int32, (_RB, k), 1)
                ).astype(jnp.float32)
        cand = jnp.where(x == m, revf, jnp.float32(0))
        idx = (jnp.float32(k) - jnp.max(cand, axis=1)).astype(jnp.int32)
        s_ref[0, 0, :] = m[:, 0] - _THRESHOLD
        i_ref[0, 0, :] = idx

    return pl.pallas_call(
        body,
        grid=(nb,),
        in_specs=[pl.BlockSpec((_RB, k), lambda i: (i, 0))],
        out_specs=[
            pl.BlockSpec((1, 1, _RB), lambda i: (i, 0, 0)),
            pl.BlockSpec((1, 1, _RB), lambda i: (i, 0, 0)),
        ],
        out_shape=[
            jax.ShapeDtypeStruct((nb, 1, _RB), jnp.float32),
            jax.ShapeDtypeStruct((nb, 1, _RB), jnp.int32),
        ],
    )


def kernel(importance, similarity, compressed_map):
    b, n, k = similarity.shape
    rows = b * n
    rows_tc = rows - _SC_ROWS
    sim2d = similarity.reshape(rows, k)

    parts_s, parts_i = [], []
    if _SC_ROWS:
        sc_s, sc_i = _make_sc_kernel(rows, k, rows_tc, _SC_ROWS)(sim2d)
    if rows_tc:
        tc_s, tc_i = _make_tc_kernel(rows // _RB, k, rows_tc // _RB)(sim2d)
        parts_s.append(tc_s.reshape(rows_tc))
        parts_i.append(tc_i.reshape(rows_tc))
    if _SC_ROWS:
        parts_s.append(sc_s)
        parts_i.append(sc_i)
    score = jnp.concatenate(parts_s) if len(parts_s) > 1 else parts_s[0]
    idx = jnp.concatenate(parts_i) if len(parts_i) > 1 else parts_i[0]
    return (score.reshape(b, n, 1), idx.reshape(b, n))


# speedup vs baseline: 1.8283x; 1.8130x over previous
import functools

import jax
import jax.numpy as jnp
from jax import lax
from jax.experimental import pallas as pl
from jax.experimental.pallas import tpu as pltpu
from jax.experimental.pallas import tpu_sc as plsc

_THRESHOLD = 0.85

_NC = 2
_NS = 16
_L = 16
_NW = _NC * _NS
_CHUNK = 64
_UNROLL = 8

_RB = 512
_SC_ROWS = 24576


@functools.lru_cache(maxsize=None)
def _make_sc_kernel(rows_all: int, k: int, row0: int, rows_sc: int):
    rows_per_w = rows_sc // _NW
    n_chunks = rows_per_w // _CHUNK
    n_slices = k // _L
    mesh = plsc.VectorSubcoreMesh(core_axis_name="c", subcore_axis_name="s")

    @functools.partial(
        pl.kernel,
        mesh=mesh,
        out_type=(
            jax.ShapeDtypeStruct((rows_sc,), jnp.float32),
            jax.ShapeDtypeStruct((rows_sc,), jnp.int32),
        ),
        scratch_types=[
            pltpu.VMEM((_CHUNK, k), jnp.float32),
            pltpu.VMEM((_CHUNK, k), jnp.float32),
            pltpu.VMEM((rows_per_w,), jnp.float32),
            pltpu.VMEM((rows_per_w,), jnp.int32),
            pltpu.VMEM((_L * 17,), jnp.float32),
            pltpu.VMEM((_L * 17,), jnp.int32),
            pltpu.SemaphoreType.DMA,
            pltpu.SemaphoreType.DMA,
        ],
        compiler_params=pltpu.CompilerParams(
            use_tc_tiling_on_sc=True, needs_layout_passes=False),
    )
    def sc_kernel(sim_hbm, score_hbm, idx_hbm,
                  buf0, buf1, acc_s, acc_i, mtx_v, mtx_i, sem0, sem1):
        wid = lax.axis_index("s") * _NC + lax.axis_index("c")
        out_base = wid * rows_per_w
        in_base = row0 + out_base
        lane = lax.iota(jnp.int32, _L)
        rowptr = lane * 17
        bufs = (buf0, buf1)
        sems = (sem0, sem1)

        def start(g, b):
            pltpu.async_copy(
                sim_hbm.at[pl.ds(in_base + g * _CHUNK, _CHUNK)],
                bufs[b], sems[b])

        def wait(b):
            pltpu.make_async_copy(
                sim_hbm.at[pl.ds(in_base, _CHUNK)], bufs[b], sems[b]).wait()

        def process(buf, g):
            def group_body(grp, carry):
                r0 = grp * _L
                for q in range(0, _L, 4):
                    vmax = [buf[r0 + q + j, pl.ds(0, _L)] for j in range(4)]
                    vidx = [lane for _ in range(4)]
                    col = lane
                    for _s in range(1, n_slices):
                        col = col + _L
                        for j in range(4):
                            v = buf[r0 + q + j, pl.ds(_s * _L, _L)]
                            pred = v > vmax[j]
                            vmax[j] = jnp.where(pred, v, vmax[j])
                            vidx[j] = jnp.where(pred, col, vidx[j])
                    for j in range(4):
                        mtx_v[pl.ds((q + j) * 17, _L)] = vmax[j]
                        mtx_i[pl.ds((q + j) * 17, _L)] = vidx[j]
                m = plsc.load_gather(mtx_v, [rowptr])
                i = plsc.load_gather(mtx_i, [rowptr])
                for c in range(1, _L):
                    vc = plsc.load_gather(mtx_v, [rowptr + c])
                    ic = plsc.load_gather(mtx_i, [rowptr + c])
                    better = (vc > m) | ((vc == m) & (ic < i))
                    m = jnp.where(better, vc, m)
                    i = jnp.where(better, ic, i)
                off = g * _CHUNK + grp * _L
                acc_s[pl.ds(off, _L)] = m - _THRESHOLD
                acc_i[pl.ds(off, _L)] = i
                return carry

            lax.fori_loop(0, _CHUNK // _L, group_body, 0)

        start(0, 0)
        start(1, 1)

        def pair_body(p, carry):
            for b in range(2):
                g = 2 * p + b
                wait(b)
                process(bufs[b], g)

                @pl.when(g + 2 < n_chunks)
                def _():
                    start(g + 2, b)
            return carry

        lax.fori_loop(0, n_chunks // 2, pair_body, 0)
        pltpu.sync_copy(acc_s, score_hbm.at[pl.ds(out_base, rows_per_w)])
        pltpu.sync_copy(acc_i, idx_hbm.at[pl.ds(out_base, rows_per_w)])

    return sc_kernel


@functools.lru_cache(maxsize=None)
def _make_tc_kernel(nb_all: int, k: int, nb: int):
    def body(x_ref, s_ref, i_ref):
        x = x_ref[...]
        m = jnp.max(x, axis=1, keepdims=True)
        revf = (k - lax.broadcasted_iota(jnp.int32, (_RB, k), 1)
                ).astype(jnp.float32)
        cand = jnp.where(x == m, revf, jnp.float32(0))
        idx = (jnp.float32(k) - jnp.max(cand, axis=1)).astype(jnp.int32)
        s_ref[0, 0, :] = m[:, 0] - _THRESHOLD
        i_ref[0, 0, :] = idx

    return pl.pallas_call(
        body,
        grid=(nb,),
        in_specs=[pl.BlockSpec((_RB, k), lambda i: (i, 0))],
        out_specs=[
            pl.BlockSpec((1, 1, _RB), lambda i: (i, 0, 0)),
            pl.BlockSpec((1, 1, _RB), lambda i: (i, 0, 0)),
        ],
        out_shape=[
            jax.ShapeDtypeStruct((nb, 1, _RB), jnp.float32),
            jax.ShapeDtypeStruct((nb, 1, _RB), jnp.int32),
        ],
    )


def kernel(importance, similarity, compressed_map):
    b, n, k = similarity.shape
    rows = b * n
    rows_tc = rows - _SC_ROWS
    sim2d = similarity.reshape(rows, k)

    parts_s, parts_i = [], []
    if _SC_ROWS:
        sc_s, sc_i = _make_sc_kernel(rows, k, rows_tc, _SC_ROWS)(sim2d)
    if rows_tc:
        tc_s, tc_i = _make_tc_kernel(rows // _RB, k, rows_tc // _RB)(sim2d)
        parts_s.append(tc_s.reshape(rows_tc))
        parts_i.append(tc_i.reshape(rows_tc))
    if _SC_ROWS:
        parts_s.append(sc_s)
        parts_i.append(sc_i)
    score = jnp.concatenate(parts_s) if len(parts_s) > 1 else parts_s[0]
    idx = jnp.concatenate(parts_i) if len(parts_i) > 1 else parts_i[0]
    return (score.reshape(b, n, 1), idx.reshape(b, n))


# speedup vs baseline: 3.2354x; 1.7696x over previous
import functools

import jax
import jax.numpy as jnp
from jax import lax
from jax.experimental import pallas as pl
from jax.experimental.pallas import tpu as pltpu
from jax.experimental.pallas import tpu_sc as plsc

_THRESHOLD = 0.85

_NC = 2
_NS = 16
_L = 16
_NW = _NC * _NS
_CHUNK = 64
_UNROLL = 8

_RB = 512
_SC_ROWS = 81920


@functools.lru_cache(maxsize=None)
def _make_sc_kernel(rows_all: int, k: int, row0: int, rows_sc: int):
    rows_per_w = rows_sc // _NW
    n_chunks = rows_per_w // _CHUNK
    n_slices = k // _L
    mesh = plsc.VectorSubcoreMesh(core_axis_name="c", subcore_axis_name="s")

    @functools.partial(
        pl.kernel,
        mesh=mesh,
        out_type=(
            jax.ShapeDtypeStruct((rows_sc,), jnp.float32),
            jax.ShapeDtypeStruct((rows_sc,), jnp.int32),
        ),
        scratch_types=[
            pltpu.VMEM((_CHUNK, k), jnp.float32),
            pltpu.VMEM((_CHUNK, k), jnp.float32),
            pltpu.VMEM((rows_per_w,), jnp.float32),
            pltpu.VMEM((rows_per_w,), jnp.int32),
            pltpu.VMEM((_L * 17,), jnp.float32),
            pltpu.VMEM((_L * 17,), jnp.int32),
            pltpu.SemaphoreType.DMA,
            pltpu.SemaphoreType.DMA,
        ],
        compiler_params=pltpu.CompilerParams(
            use_tc_tiling_on_sc=True, needs_layout_passes=False),
    )
    def sc_kernel(sim_hbm, score_hbm, idx_hbm,
                  buf0, buf1, acc_s, acc_i, mtx_v, mtx_i, sem0, sem1):
        wid = lax.axis_index("s") * _NC + lax.axis_index("c")
        out_base = wid * rows_per_w
        in_base = row0 + out_base
        lane = lax.iota(jnp.int32, _L)
        rowptr = lane * 17
        bufs = (buf0, buf1)
        sems = (sem0, sem1)

        def start(g, b):
            pltpu.async_copy(
                sim_hbm.at[pl.ds(in_base + g * _CHUNK, _CHUNK)],
                bufs[b], sems[b])

        def wait(b):
            pltpu.make_async_copy(
                sim_hbm.at[pl.ds(in_base, _CHUNK)], bufs[b], sems[b]).wait()

        def process(buf, g):
            def group_body(grp, carry):
                r0 = grp * _L
                for q in range(0, _L, 4):
                    vmax = [buf[r0 + q + j, pl.ds(0, _L)] for j in range(4)]
                    vidx = [lane for _ in range(4)]
                    col = lane
                    for _s in range(1, n_slices):
                        col = col + _L
                        for j in range(4):
                            v = buf[r0 + q + j, pl.ds(_s * _L, _L)]
                            pred = v > vmax[j]
                            vmax[j] = jnp.where(pred, v, vmax[j])
                            vidx[j] = jnp.where(pred, col, vidx[j])
                    for j in range(4):
                        mtx_v[pl.ds((q + j) * 17, _L)] = vmax[j]
                        mtx_i[pl.ds((q + j) * 17, _L)] = vidx[j]
                m = plsc.load_gather(mtx_v, [rowptr])
                i = plsc.load_gather(mtx_i, [rowptr])
                for c in range(1, _L):
                    vc = plsc.load_gather(mtx_v, [rowptr + c])
                    ic = plsc.load_gather(mtx_i, [rowptr + c])
                    better = (vc > m) | ((vc == m) & (ic < i))
                    m = jnp.where(better, vc, m)
                    i = jnp.where(better, ic, i)
                off = g * _CHUNK + grp * _L
                acc_s[pl.ds(off, _L)] = m - _THRESHOLD
                acc_i[pl.ds(off, _L)] = i
                return carry

            lax.fori_loop(0, _CHUNK // _L, group_body, 0)

        start(0, 0)
        start(1, 1)

        def pair_body(p, carry):
            for b in range(2):
                g = 2 * p + b
                wait(b)
                process(bufs[b], g)

                @pl.when(g + 2 < n_chunks)
                def _():
                    start(g + 2, b)
            return carry

        lax.fori_loop(0, n_chunks // 2, pair_body, 0)
        pltpu.sync_copy(acc_s, score_hbm.at[pl.ds(out_base, rows_per_w)])
        pltpu.sync_copy(acc_i, idx_hbm.at[pl.ds(out_base, rows_per_w)])

    return sc_kernel


@functools.lru_cache(maxsize=None)
def _make_tc_kernel(nb_all: int, k: int, nb: int):
    def body(x_ref, s_ref, i_ref):
        x = x_ref[...]
        m = jnp.max(x, axis=1, keepdims=True)
        revf = (k - lax.broadcasted_iota(jnp.int32, (_RB, k), 1)
                ).astype(jnp.float32)
        cand = jnp.where(x == m, revf, jnp.float32(0))
        idx = (jnp.float32(k) - jnp.max(cand, axis=1)).astype(jnp.int32)
        s_ref[0, 0, :] = m[:, 0] - _THRESHOLD
        i_ref[0, 0, :] = idx

    return pl.pallas_call(
        body,
        grid=(nb,),
        in_specs=[pl.BlockSpec((_RB, k), lambda i: (i, 0))],
        out_specs=[
            pl.BlockSpec((1, 1, _RB), lambda i: (i, 0, 0)),
            pl.BlockSpec((1, 1, _RB), lambda i: (i, 0, 0)),
        ],
        out_shape=[
            jax.ShapeDtypeStruct((nb, 1, _RB), jnp.float32),
            jax.ShapeDtypeStruct((nb, 1, _RB), jnp.int32),
        ],
    )


def kernel(importance, similarity, compressed_map):
    b, n, k = similarity.shape
    rows = b * n
    rows_tc = rows - _SC_ROWS
    sim2d = similarity.reshape(rows, k)

    parts_s, parts_i = [], []
    if _SC_ROWS:
        sc_s, sc_i = _make_sc_kernel(rows, k, rows_tc, _SC_ROWS)(sim2d)
    if rows_tc:
        tc_s, tc_i = _make_tc_kernel(rows // _RB, k, rows_tc // _RB)(sim2d)
        parts_s.append(tc_s.reshape(rows_tc))
        parts_i.append(tc_i.reshape(rows_tc))
    if _SC_ROWS:
        parts_s.append(sc_s)
        parts_i.append(sc_i)
    score = jnp.concatenate(parts_s) if len(parts_s) > 1 else parts_s[0]
    idx = jnp.concatenate(parts_i) if len(parts_i) > 1 else parts_i[0]
    return (score.reshape(b, n, 1), idx.reshape(b, n))


# speedup vs baseline: 3.6180x; 1.1182x over previous
import functools

import jax
import jax.numpy as jnp
from jax import lax
from jax.experimental import pallas as pl
from jax.experimental.pallas import tpu as pltpu
from jax.experimental.pallas import tpu_sc as plsc

_THRESHOLD = 0.85

_NC = 2
_NS = 16
_L = 16
_NW = _NC * _NS
_CHUNK = 64
_UNROLL = 8

_RB = 512
_SC_ROWS = 94208


@functools.lru_cache(maxsize=None)
def _make_sc_kernel(rows_all: int, k: int, row0: int, rows_sc: int):
    rows_per_w = rows_sc // _NW
    n_chunks = rows_per_w // _CHUNK
    n_slices = k // _L
    mesh = plsc.VectorSubcoreMesh(core_axis_name="c", subcore_axis_name="s")

    @functools.partial(
        pl.kernel,
        mesh=mesh,
        out_type=(
            jax.ShapeDtypeStruct((rows_sc,), jnp.float32),
            jax.ShapeDtypeStruct((rows_sc,), jnp.int32),
        ),
        scratch_types=[
            pltpu.VMEM((_CHUNK, k), jnp.float32),
            pltpu.VMEM((_CHUNK, k), jnp.float32),
            pltpu.VMEM((rows_per_w,), jnp.float32),
            pltpu.VMEM((rows_per_w,), jnp.int32),
            pltpu.VMEM((_L * 17,), jnp.float32),
            pltpu.VMEM((_L * 17,), jnp.int32),
            pltpu.SemaphoreType.DMA,
            pltpu.SemaphoreType.DMA,
        ],
        compiler_params=pltpu.CompilerParams(
            use_tc_tiling_on_sc=True, needs_layout_passes=False),
    )
    def sc_kernel(sim_hbm, score_hbm, idx_hbm,
                  buf0, buf1, acc_s, acc_i, mtx_v, mtx_i, sem0, sem1):
        wid = lax.axis_index("s") * _NC + lax.axis_index("c")
        out_base = wid * rows_per_w
        in_base = row0 + out_base
        lane = lax.iota(jnp.int32, _L)
        rowptr = lane * 17
        bufs = (buf0, buf1)
        sems = (sem0, sem1)

        def start(g, b):
            pltpu.async_copy(
                sim_hbm.at[pl.ds(in_base + g * _CHUNK, _CHUNK)],
                bufs[b], sems[b])

        def wait(b):
            pltpu.make_async_copy(
                sim_hbm.at[pl.ds(in_base, _CHUNK)], bufs[b], sems[b]).wait()

        def process(buf, g):
            def group_body(grp, carry):
                r0 = grp * _L
                for q in range(0, _L, 4):
                    vmax = [buf[r0 + q + j, pl.ds(0, _L)] for j in range(4)]
                    vidx = [lane for _ in range(4)]
                    col = lane
                    for _s in range(1, n_slices):
                        col = col + _L
                        for j in range(4):
                            v = buf[r0 + q + j, pl.ds(_s * _L, _L)]
                            pred = v > vmax[j]
                            vmax[j] = jnp.where(pred, v, vmax[j])
                            vidx[j] = jnp.where(pred, col, vidx[j])
                    for j in range(4):
                        mtx_v[pl.ds((q + j) * 17, _L)] = vmax[j]
                        mtx_i[pl.ds((q + j) * 17, _L)] = vidx[j]
                m = plsc.load_gather(mtx_v, [rowptr])
                i = plsc.load_gather(mtx_i, [rowptr])
                for c in range(1, _L):
                    vc = plsc.load_gather(mtx_v, [rowptr + c])
                    ic = plsc.load_gather(mtx_i, [rowptr + c])
                    better = (vc > m) | ((vc == m) & (ic < i))
                    m = jnp.where(better, vc, m)
                    i = jnp.where(better, ic, i)
                off = g * _CHUNK + grp * _L
                acc_s[pl.ds(off, _L)] = m - _THRESHOLD
                acc_i[pl.ds(off, _L)] = i
                return carry

            lax.fori_loop(0, _CHUNK // _L, group_body, 0)

        start(0, 0)
        start(1, 1)

        def pair_body(p, carry):
            for b in range(2):
                g = 2 * p + b
                wait(b)
                process(bufs[b], g)

                @pl.when(g + 2 < n_chunks)
                def _():
                    start(g + 2, b)
            return carry

        lax.fori_loop(0, n_chunks // 2, pair_body, 0)
        pltpu.sync_copy(acc_s, score_hbm.at[pl.ds(out_base, rows_per_w)])
        pltpu.sync_copy(acc_i, idx_hbm.at[pl.ds(out_base, rows_per_w)])

    return sc_kernel


@functools.lru_cache(maxsize=None)
def _make_tc_kernel(nb_all: int, k: int, nb: int):
    def body(x_ref, s_ref, i_ref):
        x = x_ref[...]
        m = jnp.max(x, axis=1, keepdims=True)
        revf = (k - lax.broadcasted_iota(jnp.int32, (_RB, k), 1)
                ).astype(jnp.float32)
        cand = jnp.where(x == m, revf, jnp.float32(0))
        idx = (jnp.float32(k) - jnp.max(cand, axis=1)).astype(jnp.int32)
        s_ref[0, 0, :] = m[:, 0] - _THRESHOLD
        i_ref[0, 0, :] = idx

    return pl.pallas_call(
        body,
        grid=(nb,),
        in_specs=[pl.BlockSpec((_RB, k), lambda i: (i, 0))],
        out_specs=[
            pl.BlockSpec((1, 1, _RB), lambda i: (i, 0, 0)),
            pl.BlockSpec((1, 1, _RB), lambda i: (i, 0, 0)),
        ],
        out_shape=[
            jax.ShapeDtypeStruct((nb, 1, _RB), jnp.float32),
            jax.ShapeDtypeStruct((nb, 1, _RB), jnp.int32),
        ],
    )


def kernel(importance, similarity, compressed_map):
    b, n, k = similarity.shape
    rows = b * n
    rows_tc = rows - _SC_ROWS
    sim2d = similarity.reshape(rows, k)

    parts_s, parts_i = [], []
    if _SC_ROWS:
        sc_s, sc_i = _make_sc_kernel(rows, k, rows_tc, _SC_ROWS)(sim2d)
    if rows_tc:
        tc_s, tc_i = _make_tc_kernel(rows // _RB, k, rows_tc // _RB)(sim2d)
        parts_s.append(tc_s.reshape(rows_tc))
        parts_i.append(tc_i.reshape(rows_tc))
    if _SC_ROWS:
        parts_s.append(sc_s)
        parts_i.append(sc_i)
    score = jnp.concatenate(parts_s) if len(parts_s) > 1 else parts_s[0]
    idx = jnp.concatenate(parts_i) if len(parts_i) > 1 else parts_i[0]
    return (score.reshape(b, n, 1), idx.reshape(b, n))
